# baseline (speedup 1.0000x reference)
import functools

import jax
import jax.numpy as jnp
from jax import lax
from jax.experimental import pallas as pl
from jax.experimental.pallas import tpu as pltpu
from jax.experimental.pallas import tpu_sc as plsc

_D = 64
_GRP = 128
_K = 5
_CHUNK = _GRP * _K
_NC = 2
_NS = 16
_NW = _NC * _NS


@functools.partial(jax.jit, static_argnums=(2,))
def _sc_gather(table, idx_flat, n):
    rows_per_w = n // _NW
    nchunks = rows_per_w // _CHUNK
    mesh = plsc.VectorSubcoreMesh(core_axis_name="c", subcore_axis_name="s")

    @functools.partial(
        pl.kernel,
        out_type=jax.ShapeDtypeStruct((n, _D), jnp.float32),
        mesh=mesh,
        scratch_types=[
            pltpu.VMEM((_CHUNK,), jnp.int32),
            pltpu.VMEM((_CHUNK, _D), jnp.float32),
            pltpu.SemaphoreType.DMA,
        ],
        compiler_params=pltpu.CompilerParams(use_tc_tiling_on_sc=False),
    )
    def k(table_hbm, idx_hbm, out_hbm, idx_v, rows_v, gsem):
        wid = lax.axis_index("s") * _NC + lax.axis_index("c")
        base = wid * rows_per_w

        def body(c, carry):
            r0 = base + c * _CHUNK
            pltpu.sync_copy(idx_hbm.at[pl.ds(r0, _CHUNK)], idx_v)
            copies = [
                pltpu.async_copy(
                    table_hbm.at[idx_v.at[pl.ds(j * _GRP, _GRP)]],
                    rows_v.at[pl.ds(j * _GRP, _GRP)],
                    gsem,
                )
                for j in range(_K)
            ]
            for cp in copies:
                cp.wait()
            pltpu.sync_copy(rows_v, out_hbm.at[pl.ds(r0, _CHUNK)])
            return carry

        lax.fori_loop(0, nchunks, body, 0)

    return k(table, idx_flat)


def kernel(input_tensor, indices):
    b, s = indices.shape
    n = b * s
    idx_flat = indices.reshape(n).astype(jnp.int32)
    out = _sc_gather(input_tensor, idx_flat, n)
    return out.reshape(b, s, _D)


# speedup vs baseline: 1.0087x; 1.0087x over previous
import functools

import jax
import jax.numpy as jnp
from jax import lax
from jax.experimental import pallas as pl
from jax.experimental.pallas import tpu as pltpu
from jax.experimental.pallas import tpu_sc as plsc

_D = 64
_GRP = 128
_K = 5
_CHUNK = _GRP * _K
_NC = 2
_NS = 16
_NW = _NC * _NS


@functools.partial(jax.jit, static_argnums=(2,))
def _sc_gather(table, idx_flat, n):
    rows_per_w = n // _NW
    nchunks = rows_per_w // _CHUNK
    mesh = plsc.VectorSubcoreMesh(core_axis_name="c", subcore_axis_name="s")

    @functools.partial(
        pl.kernel,
        out_type=jax.ShapeDtypeStruct((n, _D), jnp.float32),
        mesh=mesh,
        scratch_types=[
            pltpu.VMEM((2, _CHUNK), jnp.int32),
            pltpu.VMEM((2, _CHUNK, _D), jnp.float32),
            pltpu.SemaphoreType.DMA,
            pltpu.SemaphoreType.DMA,
            pltpu.SemaphoreType.DMA,
            pltpu.SemaphoreType.DMA,
        ],
        compiler_params=pltpu.CompilerParams(use_tc_tiling_on_sc=False),
    )
    def k(table_hbm, idx_hbm, out_hbm, idx_v, rows_v, g0, g1, s0, s1):
        wid = lax.axis_index("s") * _NC + lax.axis_index("c")
        base = wid * rows_per_w
        gsem = (g0, g1)
        ssem = (s0, s1)

        def load_idx(c, b):
            pltpu.sync_copy(idx_hbm.at[pl.ds(base + c * _CHUNK, _CHUNK)],
                            idx_v.at[b])

        def fire_gathers(c, b):
            for j in range(_K):
                pltpu.async_copy(
                    table_hbm.at[idx_v.at[b, pl.ds(j * _GRP, _GRP)]],
                    rows_v.at[b, pl.ds(j * _GRP, _GRP)],
                    gsem[b],
                )

        def wait_gathers(b):
            for j in range(_K):
                pltpu.make_async_copy(
                    table_hbm.at[idx_v.at[b, pl.ds(j * _GRP, _GRP)]],
                    rows_v.at[b, pl.ds(j * _GRP, _GRP)],
                    gsem[b],
                ).wait()

        def fire_scatter(c, b):
            pltpu.async_copy(rows_v.at[b],
                             out_hbm.at[pl.ds(base + c * _CHUNK, _CHUNK)],
                             ssem[b])

        def wait_scatter(c, b):
            pltpu.make_async_copy(rows_v.at[b],
                                  out_hbm.at[pl.ds(base + c * _CHUNK, _CHUNK)],
                                  ssem[b]).wait()

        for c in range(nchunks):
            b = c % 2
            if c >= 2:
                wait_scatter(c - 2, b)
            load_idx(c, b)
            fire_gathers(c, b)
            if c >= 1:
                wait_gathers(1 - b)
                fire_scatter(c - 1, 1 - b)
        last = nchunks - 1
        wait_gathers(last % 2)
        fire_scatter(last, last % 2)
        if nchunks >= 2:
            wait_scatter(last - 1, (last - 1) % 2)
        wait_scatter(last, last % 2)

    return k(table, idx_flat)


def kernel(input_tensor, indices):
    b, s = indices.shape
    n = b * s
    idx_flat = indices.reshape(n).astype(jnp.int32)
    out = _sc_gather(input_tensor, idx_flat, n)
    return out.reshape(b, s, _D)
